# baseline (speedup 1.0000x reference)
import functools

import jax
import jax.numpy as jnp
from jax import lax
from jax.experimental import pallas as pl
from jax.experimental.pallas import tpu as pltpu
from jax.experimental.pallas import tpu_sc as plsc

N_SIDE = 5000
D = 128
E_SIDE = 160000
NC, NS, L = 2, 16, 16
NP = 5120
RPT = NP // NS
K = 128
EP = 163840
CH = EP // (NS * K)
CHH = CH // 2
NQ = RPT // 64

_f32 = jnp.float32
_i32 = jnp.int32

_mesh = plsc.VectorSubcoreMesh(core_axis_name="c", subcore_axis_name="s",
                               num_cores=NC, num_subcores=NS)

_SCRATCH = [
    pltpu.VMEM_SHARED((NP, D), _f32),
    pltpu.VMEM_SHARED((NP, D), _f32),
    pltpu.VMEM((CHH, K), _i32),
    pltpu.VMEM((CHH, K), _i32),
    pltpu.VMEM((K, D), _f32),
    pltpu.VMEM((K, D), _f32),
    pltpu.SemaphoreType.DMA,
]


def _nr_rsqrt(x):
    i = lax.bitcast_convert_type(x, _i32)
    i = jnp.int32(0x5F3759DF) - lax.shift_right_logical(i, 1)
    y = lax.bitcast_convert_type(i, _f32)
    for _ in range(3):
        y = y * (jnp.float32(1.5) - jnp.float32(0.5) * x * y * y)
    return y


def _sum_loop(tab_s, acc, src2, dst2, srcv, dstv, rows0, rows1, sem, c, s):
    for h in range(2):
        pltpu.sync_copy(src2.at[c, s, pl.ds(h * CHH, CHH)], srcv)
        pltpu.sync_copy(dst2.at[c, s, pl.ds(h * CHH, CHH)], dstv)
        pltpu.async_copy(tab_s.at[srcv.at[0]], rows0, sem)

        def body(i, carry):
            j = 2 * i
            pltpu.make_async_copy(tab_s.at[srcv.at[j]], rows0, sem).wait()
            pltpu.async_copy(tab_s.at[srcv.at[j + 1]], rows1, sem)
            pltpu.sync_copy(rows0, acc.at[dstv.at[j]], add=True)
            pltpu.make_async_copy(tab_s.at[srcv.at[j]], rows1, sem).wait()

            @pl.when(j + 2 < CHH)
            def _():
                pltpu.async_copy(tab_s.at[srcv.at[j + 2]], rows0, sem)

            pltpu.sync_copy(rows1, acc.at[dstv.at[j + 1]], add=True)
            return carry

        lax.fori_loop(0, CHH // 2, body, 0)


@functools.partial(
    pl.kernel,
    out_type=[jax.ShapeDtypeStruct((NC, NP, D), _f32),
              jax.ShapeDtypeStruct((NC, NP, D), _f32)],
    mesh=_mesh,
    scratch_types=_SCRATCH,
)
def _layer1_sc(z2, src2, dst2, ones_hbm, zero_hbm, t1out, degout,
               acc, tab_s, srcv, dstv, rows0, rows1, sem):
    c = lax.axis_index("c")
    s = lax.axis_index("s")
    sl_own = pl.ds(s * RPT, RPT)
    pltpu.sync_copy(zero_hbm, acc.at[sl_own])
    pltpu.sync_copy(zero_hbm, tab_s.at[sl_own])
    pltpu.sync_copy(ones_hbm, rows0)
    plsc.subcore_barrier()

    for h in range(2):
        pltpu.sync_copy(dst2.at[1 - c, s, pl.ds(h * CHH, CHH)], dstv)

        def dbody(j, carry):
            pltpu.sync_copy(rows0, tab_s.at[dstv.at[j]], add=True)
            return carry

        lax.fori_loop(0, CHH, dbody, 0)

    plsc.subcore_barrier()

    def stage(q, carry):
        base = s * RPT + q * 64
        pltpu.sync_copy(tab_s.at[pl.ds(base, 64)], rows0.at[pl.ds(0, 64)])
        pltpu.sync_copy(z2.at[c, pl.ds(base, 64)], rows1.at[pl.ds(0, 64)])
        pltpu.sync_copy(rows0.at[pl.ds(0, 64)], degout.at[c, pl.ds(base, 64)])

        def row(r, carry2):
            d16 = rows0[r, pl.ds(0, 16)]
            dis = jnp.where(d16 > 0, _nr_rsqrt(d16), jnp.float32(0.0))
            for q2 in range(8):
                sl = pl.ds(16 * q2, 16)
                rows1[r, sl] = rows1[r, sl] * dis
            return carry2

        lax.fori_loop(0, 64, row, 0)
        pltpu.sync_copy(rows1.at[pl.ds(0, 64)], tab_s.at[pl.ds(base, 64)])
        return carry

    lax.fori_loop(0, NQ, stage, 0)
    plsc.subcore_barrier()

    _sum_loop(tab_s, acc, src2, dst2, srcv, dstv, rows0, rows1, sem, c, s)
    plsc.subcore_barrier()
    pltpu.sync_copy(acc.at[sl_own], t1out.at[c, sl_own])


@functools.partial(
    pl.kernel,
    out_type=jax.ShapeDtypeStruct((NC, NP, D), _f32),
    mesh=_mesh,
    scratch_types=_SCRATCH,
)
def _layer2_sc(t1_2, deg2, z2, src2, dst2, zero_hbm, out,
               acc, tab_s, srcv, dstv, rows0, rows1, sem):
    c = lax.axis_index("c")
    s = lax.axis_index("s")
    sl_own = pl.ds(s * RPT, RPT)
    pltpu.sync_copy(zero_hbm, acc.at[sl_own])

    def stage(q, carry):
        base = s * RPT + q * 64
        pltpu.sync_copy(t1_2.at[1 - c, pl.ds(base, 64)], rows0.at[pl.ds(0, 64)])
        pltpu.sync_copy(deg2.at[c, pl.ds(base, 64)], rows1.at[pl.ds(0, 64)])

        def row(r, carry2):
            d16 = rows1[r, pl.ds(0, 16)]
            y = _nr_rsqrt(d16)
            rec = jnp.where(d16 > 0, y * y, jnp.float32(0.0))
            for q2 in range(8):
                sl = pl.ds(16 * q2, 16)
                rows0[r, sl] = rows0[r, sl] * rec
            return carry2

        lax.fori_loop(0, 64, row, 0)
        pltpu.sync_copy(rows0.at[pl.ds(0, 64)], tab_s.at[pl.ds(base, 64)])
        return carry

    lax.fori_loop(0, NQ, stage, 0)
    plsc.subcore_barrier()

    _sum_loop(tab_s, acc, src2, dst2, srcv, dstv, rows0, rows1, sem, c, s)
    plsc.subcore_barrier()

    def drain(q, carry):
        base = s * RPT + q * 64
        pltpu.sync_copy(acc.at[pl.ds(base, 64)], rows0.at[pl.ds(0, 64)])
        pltpu.sync_copy(t1_2.at[c, pl.ds(base, 64)], rows0.at[pl.ds(64, 64)])
        pltpu.sync_copy(z2.at[1 - c, pl.ds(base, 64)], rows1.at[pl.ds(0, 64)])
        pltpu.sync_copy(deg2.at[1 - c, pl.ds(base, 64)], rows1.at[pl.ds(64, 64)])

        def row(r, carry2):
            d16 = rows1[r + 64, pl.ds(0, 16)]
            dis = jnp.where(d16 > 0, _nr_rsqrt(d16), jnp.float32(0.0))
            for q2 in range(8):
                sl = pl.ds(16 * q2, 16)
                t2v = rows0[r, sl]
                t1v = rows0[r + 64, sl]
                zv = rows1[r, sl]
                rows0[r, sl] = (zv + dis * (t1v + t2v)) * jnp.float32(1.0 / 3.0)
            return carry2

        lax.fori_loop(0, 64, row, 0)
        pltpu.sync_copy(rows0.at[pl.ds(0, 64)], out.at[c, pl.ds(base, 64)])
        return carry

    lax.fori_loop(0, NQ, drain, 0)


def _proj_tc(xu, wu, bu, xi, wi, bi, zu, zi):
    dn = (((1,), (1,)), ((), ()))
    zu[...] = lax.dot_general(xu[...], wu[...], dn,
                              preferred_element_type=_f32) + bu[...]
    zi[...] = lax.dot_general(xi[...], wi[...], dn,
                              preferred_element_type=_f32) + bi[...]


_sds = jax.ShapeDtypeStruct
_proj_call = pl.pallas_call(_proj_tc, out_shape=[_sds((NP, D), _f32)] * 2)


def kernel(x_user, x_item, edge_index_user_item, edge_index_item_user,
           W_user, b_user, W_item, b_item):
    padn = NP - N_SIDE
    pade = EP - E_SIDE
    xu = jnp.pad(x_user, ((0, padn), (0, 0)))
    xi = jnp.pad(x_item, ((0, padn), (0, 0)))

    def edges4(a, b, fa, fb):
        a = jnp.concatenate([a, jnp.full((pade,), fa, _i32)]).reshape(NS, CH, K)
        b = jnp.concatenate([b, jnp.full((pade,), fb, _i32)]).reshape(NS, CH, K)
        return jnp.stack([a, b], axis=0)

    src2 = edges4(edge_index_user_item[0], edge_index_item_user[0], 0, 0)
    dst2 = edges4(edge_index_user_item[1], edge_index_item_user[1],
                  NP - 1, NP - 1)

    onesK = jnp.ones((K, D), _f32)
    zeroD = jnp.zeros((RPT, D), _f32)

    zu, zi = _proj_call(xu, W_user, b_user.reshape(1, D),
                        xi, W_item, b_item.reshape(1, D))
    z2 = jnp.stack([zu, zi])

    t1_2, deg2 = _layer1_sc(z2, src2, dst2, onesK, zeroD)
    f2 = _layer2_sc(t1_2, deg2, z2, src2, dst2, zeroD)
    return (f2[1, :N_SIDE], f2[0, :N_SIDE])
